# baseline (speedup 1.0000x reference)
import functools

import jax
import jax.numpy as jnp
from jax.experimental import pallas as pl
from jax.experimental.pallas import tpu as pltpu

BI = 32
P = 4


def _silu(x):
    h = 0.5 * x
    return h * jnp.tanh(h) + h


def _selu(x):
    alpha = 1.6732632423543772
    scale = 1.0507009873554805
    return scale * jnp.where(x > 0, x, alpha * (jnp.exp(x) - 1.0))


def _egnn_step(
    fn_ref, fp_ref, cn_ref, cp_ref, v_ref, e_ref,
    whi_ref, whjb_ref, be1_ref, wd_ref, webs_ref,
    w2b_ref, be2t_ref, wx1b_ref, bx1t_ref, wx2e_ref, bx2_ref,
    wv_ref, bv_ref, wh1a_ref, wh1b_ref, bh1_ref, wh2_ref, bh2_ref,
    msfold_ref, agfold_ref,
    hout_ref, cout_ref, vout_ref,
    hi_scr, hj_scr,
    *, n, d, h, m,
):
    i = pl.program_id(0)
    g8 = webs_ref.shape[0]
    nq = n // (P * g8)
    r4 = g8 * BI * nq
    ph, pm, p3 = P * h, P * m, P * 3

    @pl.when(i == 0)
    def _():
        hi_scr[...] = (
            jnp.dot(fn_ref[...], whi_ref[...],
                    preferred_element_type=jnp.float32)
            + be1_ref[...]
        )
        hj_scr[...] = jnp.dot(fp_ref[...], whjb_ref[...],
                              preferred_element_type=jnp.float32)

    base = i * BI
    fi = fn_ref[pl.ds(base, BI), :]
    ci = cn_ref[pl.ds(base, BI), :]
    vi = v_ref[...]

    hi_blk = hi_scr[pl.ds(base, BI), :]
    hi_tile = jnp.concatenate([hi_blk] * P, axis=1)

    ci_tile = jnp.concatenate([ci] * P, axis=1)
    rel = (ci_tile[None, :, None, :]
           - cp_ref[...].reshape(g8, 1, nq, p3)).reshape(r4, p3)
    rel2 = rel * rel

    et = e_ref[...].reshape(BI * nq, e_ref.shape[-1])
    e_term = jnp.concatenate(
        [jnp.dot(et, webs_ref[g], preferred_element_type=jnp.float32)
         for g in range(g8)],
        axis=0,
    )

    x0 = (
        (jnp.dot(rel2, wd_ref[...], preferred_element_type=jnp.float32)
         + e_term).reshape(g8, BI, nq, ph)
        + hi_tile[None, :, None, :]
        + hj_scr[...].reshape(g8, 1, nq, ph)
    ).reshape(r4, ph)
    t1 = _silu(x0)
    mf = _silu(
        jnp.dot(t1, w2b_ref[...], preferred_element_type=jnp.float32)
        + be2t_ref[...]
    )
    g = _silu(
        jnp.dot(mf, wx1b_ref[...], preferred_element_type=jnp.float32)
        + bx1t_ref[...]
    )
    cw12 = (
        jnp.dot(g, wx2e_ref[...], preferred_element_type=jnp.float32)
        + bx2_ref[...]
    )

    w_rel = rel * cw12
    agg12 = jnp.sum(w_rel.reshape(g8, BI, nq, p3), axis=(0, 2))
    agg = jnp.dot(agg12, agfold_ref[...],
                  preferred_element_type=jnp.float32) / (n - 1)

    ms128 = jnp.sum(mf.reshape(g8, BI, nq, pm), axis=(0, 2))
    msum = jnp.dot(ms128, msfold_ref[...],
                   preferred_element_type=jnp.float32)

    vscale = (
        jnp.dot(fi, wv_ref[...], preferred_element_type=jnp.float32)
        + bv_ref[...]
    )
    vnew = vscale * vi + agg
    cnew = ci + vnew

    hpre = _silu(
        jnp.dot(fi, wh1a_ref[...], preferred_element_type=jnp.float32)
        + jnp.dot(msum, wh1b_ref[...], preferred_element_type=jnp.float32)
        + bh1_ref[...]
    )
    hout_ref[...] = (
        fi
        + jnp.dot(hpre, wh2_ref[...], preferred_element_type=jnp.float32)
        + bh2_ref[...]
    )
    cout_ref[...] = _selu(cnew)
    vout_ref[...] = _selu(vnew)


@jax.jit
def kernel(feats, coors, vel, edges, We1, be1, We2, be2, Wx1, bx1, Wx2, bx2,
           Wv, bv, Wh1, bh1, Wh2, bh2):
    b, n, d = feats.shape
    ed = edges.shape[-1]
    h = We1.shape[1]
    m = We2.shape[1]
    np4 = n // P
    g8 = 128 // (P * ed)
    nq = n // (P * g8)

    fn = feats[0]
    cn = coors[0]
    v2 = vel[0]
    fp = (feats[0].reshape(nq, g8, P * d)
          .transpose(1, 0, 2).reshape(np4, P * d))
    cp = (coors[0].reshape(nq, g8, P * 3)
          .transpose(1, 0, 2).reshape(np4, P * 3))
    ep = edges[0].reshape(n * nq, g8 * P * ed)

    eyep = jnp.eye(P, dtype=jnp.float32)
    onesp = jnp.ones((P, 1), dtype=jnp.float32)

    whi = We1[:d]
    whjb = jnp.kron(eyep, We1[d:2 * d])
    wd2 = We1[2 * d:2 * d + 1]
    wd = jnp.kron(eyep, jnp.ones((3, 1), jnp.float32) @ wd2)
    web = jnp.kron(eyep, We1[2 * d + 1:])
    webs = jnp.stack([
        jnp.pad(web, ((g * P * ed, (g8 - 1 - g) * P * ed), (0, 0)))
        for g in range(g8)
    ])
    w2b = jnp.kron(eyep, We2)
    wx1b = jnp.kron(eyep, Wx1)
    wx2e = jnp.kron(eyep, Wx2 @ jnp.ones((1, 3), jnp.float32))
    wh1a = Wh1[:d]
    wh1b = Wh1[d:]
    msfold = jnp.kron(onesp, jnp.eye(m, dtype=jnp.float32))
    agfold = jnp.kron(onesp, jnp.eye(3, dtype=jnp.float32))

    be1r = be1.reshape(1, h)
    be2t = jnp.concatenate([be2.reshape(1, m)] * P, axis=1)
    bx1t = jnp.concatenate([bx1.reshape(1, h)] * P, axis=1)
    bx2r = bx2.reshape(1, 1)
    bvr = bv.reshape(1, 1)
    bh1r = bh1.reshape(1, h)
    bh2r = bh2.reshape(1, d)

    full = lambda shape: pl.BlockSpec(shape, lambda i: tuple(0 for _ in shape))

    grid = (n // BI,)
    out = pl.pallas_call(
        functools.partial(_egnn_step, n=n, d=d, h=h, m=m),
        grid=grid,
        in_specs=[
            full((n, d)),
            full((np4, P * d)),
            full((n, 3)),
            full((np4, P * 3)),
            pl.BlockSpec((BI, 3), lambda i: (i, 0)),
            pl.BlockSpec((BI * nq, g8 * P * ed), lambda i: (i, 0)),
            full((d, h)), full((P * d, P * h)), full((1, h)),
            full((P * 3, P * h)), full((g8, P * ed * g8, P * h)),
            full((P * h, P * m)), full((1, P * m)),
            full((P * m, P * h)), full((1, P * h)),
            full((P * h, P * 3)), full((1, 1)),
            full((d, 1)), full((1, 1)),
            full((d, h)), full((m, h)), full((1, h)), full((h, d)),
            full((1, d)),
            full((P * m, m)), full((P * 3, 3)),
        ],
        out_specs=[
            pl.BlockSpec((BI, d), lambda i: (i, 0)),
            pl.BlockSpec((BI, 3), lambda i: (i, 0)),
            pl.BlockSpec((BI, 3), lambda i: (i, 0)),
        ],
        out_shape=[
            jax.ShapeDtypeStruct((n, d), jnp.float32),
            jax.ShapeDtypeStruct((n, 3), jnp.float32),
            jax.ShapeDtypeStruct((n, 3), jnp.float32),
        ],
        scratch_shapes=[
            pltpu.VMEM((n, h), jnp.float32),
            pltpu.VMEM((np4, P * h), jnp.float32),
        ],
        compiler_params=pltpu.CompilerParams(
            dimension_semantics=("arbitrary",),
        ),
    )(fn, fp, cn, cp, v2, ep,
      whi, whjb, be1r, wd, webs,
      w2b, be2t, wx1b, bx1t, wx2e, bx2r,
      Wv, bvr, wh1a, wh1b, bh1r, Wh2, bh2r,
      msfold, agfold)

    h_out, c_out, v_out = out
    return (h_out[None], c_out[None], v_out[None])


# speedup vs baseline: 7.2319x; 7.2319x over previous
import functools

import jax
import jax.numpy as jnp
from jax.experimental import pallas as pl
from jax.experimental.pallas import tpu as pltpu

BI = 32
P = 4


def _silu(x):
    h = 0.5 * x
    return h * jnp.tanh(h) + h


def _selu(x):
    alpha = 1.6732632423543772
    scale = 1.0507009873554805
    return scale * jnp.where(x > 0, x, alpha * (jnp.exp(x) - 1.0))


def _egnn_step(
    fn_ref, fp_ref, cn_ref, cp_ref, v_ref, e_ref,
    whi_ref, whjb_ref, be1_ref, wd_ref, webs_ref,
    w2b_ref, be2t_ref, wx1b_ref, bx1t_ref, wx2e_ref, bx2_ref,
    wv_ref, bv_ref, wh1a_ref, wh1b_ref, bh1_ref, wh2_ref, bh2_ref,
    msfold_ref, agfold_ref,
    hout_ref, cout_ref, vout_ref,
    hi_scr, hj_scr,
    *, n, d, h, m,
):
    i = pl.program_id(0)
    g8 = webs_ref.shape[0]
    nq = n // (P * g8)
    r4 = g8 * BI * nq
    ph, pm, p3 = P * h, P * m, P * 3

    @pl.when(i == 0)
    def _():
        hi_scr[...] = (
            jnp.dot(fn_ref[...], whi_ref[...],
                    preferred_element_type=jnp.float32)
            + be1_ref[...]
        )
        hj_scr[...] = jnp.dot(fp_ref[...], whjb_ref[...],
                              preferred_element_type=jnp.float32)

    base = i * BI
    fi = fn_ref[pl.ds(base, BI), :]
    ci = cn_ref[pl.ds(base, BI), :]
    vi = v_ref[...]

    hi_blk = hi_scr[pl.ds(base, BI), :]
    hi_tile = jnp.concatenate([hi_blk] * P, axis=1)

    ci_tile = jnp.concatenate([ci] * P, axis=1)
    rel = (ci_tile[None, :, None, :]
           - cp_ref[...].reshape(g8, 1, nq, p3)).reshape(r4, p3)
    rel2 = rel * rel

    et = e_ref[...].reshape(BI * nq, e_ref.shape[-1])
    e_term = jnp.concatenate(
        [jnp.dot(et, webs_ref[g], preferred_element_type=jnp.float32)
         for g in range(g8)],
        axis=0,
    )

    x0 = (
        (jnp.dot(rel2, wd_ref[...], preferred_element_type=jnp.float32)
         + e_term).reshape(g8, BI, nq, ph)
        + hi_tile[None, :, None, :]
        + hj_scr[...].reshape(g8, 1, nq, ph)
    ).reshape(r4, ph)
    t1 = _silu(x0)
    mf = _silu(
        jnp.dot(t1, w2b_ref[...], preferred_element_type=jnp.float32)
        + be2t_ref[...]
    )
    g = _silu(
        jnp.dot(mf, wx1b_ref[...], preferred_element_type=jnp.float32)
        + bx1t_ref[...]
    )
    cw12 = (
        jnp.dot(g, wx2e_ref[...], preferred_element_type=jnp.float32)
        + bx2_ref[...]
    )

    w_rel = rel * cw12
    agg12 = jnp.sum(w_rel.reshape(g8, BI, nq, p3), axis=(0, 2))
    agg = jnp.dot(agg12, agfold_ref[...],
                  preferred_element_type=jnp.float32) / (n - 1)

    ms128 = jnp.sum(mf.reshape(g8, BI, nq, pm), axis=(0, 2))
    msum = jnp.dot(ms128, msfold_ref[...],
                   preferred_element_type=jnp.float32)

    vscale = (
        jnp.dot(fi, wv_ref[...], preferred_element_type=jnp.float32)
        + bv_ref[...]
    )
    vnew = vscale * vi + agg
    cnew = ci + vnew

    hpre = _silu(
        jnp.dot(fi, wh1a_ref[...], preferred_element_type=jnp.float32)
        + jnp.dot(msum, wh1b_ref[...], preferred_element_type=jnp.float32)
        + bh1_ref[...]
    )
    hout_ref[...] = (
        fi
        + jnp.dot(hpre, wh2_ref[...], preferred_element_type=jnp.float32)
        + bh2_ref[...]
    )
    cout_ref[...] = _selu(cnew)
    vout_ref[...] = _selu(vnew)


@jax.jit
def kernel(feats, coors, vel, edges, We1, be1, We2, be2, Wx1, bx1, Wx2, bx2,
           Wv, bv, Wh1, bh1, Wh2, bh2):
    b, n, d = feats.shape
    ed = edges.shape[-1]
    h = We1.shape[1]
    m = We2.shape[1]
    np4 = n // P
    g8 = 128 // (P * ed)
    nq = n // (P * g8)

    fn = feats[0]
    cn = coors[0]
    v2 = vel[0]
    fp = (feats[0].reshape(nq, g8, P * d)
          .transpose(1, 0, 2).reshape(np4, P * d))
    cp = (coors[0].reshape(nq, g8, P * 3)
          .transpose(1, 0, 2).reshape(np4, P * 3))
    ep = edges[0].reshape(n, nq, g8 * P * ed)

    eyep = jnp.eye(P, dtype=jnp.float32)
    onesp = jnp.ones((P, 1), dtype=jnp.float32)

    whi = We1[:d]
    whjb = jnp.kron(eyep, We1[d:2 * d])
    wd2 = We1[2 * d:2 * d + 1]
    wd = jnp.kron(eyep, jnp.ones((3, 1), jnp.float32) @ wd2)
    web = jnp.kron(eyep, We1[2 * d + 1:])
    webs = jnp.stack([
        jnp.pad(web, ((g * P * ed, (g8 - 1 - g) * P * ed), (0, 0)))
        for g in range(g8)
    ])
    w2b = jnp.kron(eyep, We2)
    wx1b = jnp.kron(eyep, Wx1)
    wx2e = jnp.kron(eyep, Wx2 @ jnp.ones((1, 3), jnp.float32))
    wh1a = Wh1[:d]
    wh1b = Wh1[d:]
    msfold = jnp.kron(onesp, jnp.eye(m, dtype=jnp.float32))
    agfold = jnp.kron(onesp, jnp.eye(3, dtype=jnp.float32))

    be1r = be1.reshape(1, h)
    be2t = jnp.concatenate([be2.reshape(1, m)] * P, axis=1)
    bx1t = jnp.concatenate([bx1.reshape(1, h)] * P, axis=1)
    bx2r = bx2.reshape(1, 1)
    bvr = bv.reshape(1, 1)
    bh1r = bh1.reshape(1, h)
    bh2r = bh2.reshape(1, d)

    full = lambda shape: pl.BlockSpec(shape, lambda i: tuple(0 for _ in shape))

    grid = (n // BI,)
    out = pl.pallas_call(
        functools.partial(_egnn_step, n=n, d=d, h=h, m=m),
        grid=grid,
        in_specs=[
            full((n, d)),
            full((np4, P * d)),
            full((n, 3)),
            full((np4, P * 3)),
            pl.BlockSpec((BI, 3), lambda i: (i, 0)),
            pl.BlockSpec((BI, nq, g8 * P * ed), lambda i: (i, 0, 0)),
            full((d, h)), full((P * d, P * h)), full((1, h)),
            full((P * 3, P * h)), full((g8, P * ed * g8, P * h)),
            full((P * h, P * m)), full((1, P * m)),
            full((P * m, P * h)), full((1, P * h)),
            full((P * h, P * 3)), full((1, 1)),
            full((d, 1)), full((1, 1)),
            full((d, h)), full((m, h)), full((1, h)), full((h, d)),
            full((1, d)),
            full((P * m, m)), full((P * 3, 3)),
        ],
        out_specs=[
            pl.BlockSpec((BI, d), lambda i: (i, 0)),
            pl.BlockSpec((BI, 3), lambda i: (i, 0)),
            pl.BlockSpec((BI, 3), lambda i: (i, 0)),
        ],
        out_shape=[
            jax.ShapeDtypeStruct((n, d), jnp.float32),
            jax.ShapeDtypeStruct((n, 3), jnp.float32),
            jax.ShapeDtypeStruct((n, 3), jnp.float32),
        ],
        scratch_shapes=[
            pltpu.VMEM((n, h), jnp.float32),
            pltpu.VMEM((np4, P * h), jnp.float32),
        ],
        compiler_params=pltpu.CompilerParams(
            dimension_semantics=("arbitrary",),
        ),
    )(fn, fp, cn, cp, v2, ep,
      whi, whjb, be1r, wd, webs,
      w2b, be2t, wx1b, bx1t, wx2e, bx2r,
      Wv, bvr, wh1a, wh1b, bh1r, Wh2, bh2r,
      msfold, agfold)

    h_out, c_out, v_out = out
    return (h_out[None], c_out[None], v_out[None])


# speedup vs baseline: 7.8170x; 1.0809x over previous
import functools

import jax
import jax.numpy as jnp
from jax.experimental import pallas as pl
from jax.experimental.pallas import tpu as pltpu

BI = 64
P = 4


def _silu(x):
    h = 0.5 * x
    return h * jnp.tanh(h) + h


def _silu_pre(h):
    return h * jnp.tanh(h) + h


def _selu(x):
    alpha = 1.6732632423543772
    scale = 1.0507009873554805
    return scale * jnp.where(x > 0, x, alpha * (jnp.exp(x) - 1.0))


def _egnn_step(
    fn_ref, fp_ref, cn_ref, cp_ref, v_ref, e_ref,
    whi_ref, whjb_ref, be1_ref, wd_ref, webs_ref,
    w2b_ref, be2t_ref, wx1b_ref, bx1t_ref, wx2e_ref, bx2_ref,
    wv_ref, bv_ref, wh1a_ref, wh1b_ref, bh1_ref, wh2_ref, bh2_ref,
    msfold_ref, agfold_ref,
    hout_ref, cout_ref, vout_ref,
    hi_scr, hj_scr,
    *, n, d, h, m,
):
    i = pl.program_id(0)
    g8 = webs_ref.shape[0]
    nq = n // (P * g8)
    r4 = g8 * BI * nq
    ph, pm, p3 = P * h, P * m, P * 3

    @pl.when(i == 0)
    def _():
        hi_scr[...] = 0.5 * (
            jnp.dot(fn_ref[...], whi_ref[...],
                    preferred_element_type=jnp.float32)
            + be1_ref[...]
        )
        hj_scr[...] = 0.5 * jnp.dot(fp_ref[...], whjb_ref[...],
                                    preferred_element_type=jnp.float32)

    base = i * BI
    fi = fn_ref[pl.ds(base, BI), :]
    ci = cn_ref[pl.ds(base, BI), :]
    vi = v_ref[...]

    hi_blk = hi_scr[pl.ds(base, BI), :]
    hi_tile = jnp.concatenate([hi_blk] * P, axis=1)

    ci_tile = jnp.concatenate([ci] * P, axis=1)
    rel = (ci_tile[None, :, None, :]
           - cp_ref[...].reshape(g8, 1, nq, p3)).reshape(r4, p3)
    rel2 = rel * rel

    et = e_ref[...].reshape(BI * nq, e_ref.shape[-1])
    e_term = jnp.concatenate(
        [jnp.dot(et, webs_ref[g], preferred_element_type=jnp.float32)
         for g in range(g8)],
        axis=0,
    )

    x0 = (
        (jnp.dot(rel2, wd_ref[...], preferred_element_type=jnp.float32)
         + e_term).reshape(g8, BI, nq, ph)
        + hi_tile[None, :, None, :]
        + hj_scr[...].reshape(g8, 1, nq, ph)
    ).reshape(r4, ph)
    t1 = _silu_pre(x0)
    mf = _silu_pre(
        jnp.dot(t1, w2b_ref[...], preferred_element_type=jnp.float32)
        + be2t_ref[...]
    )
    g = _silu_pre(
        jnp.dot(mf, wx1b_ref[...], preferred_element_type=jnp.float32)
        + bx1t_ref[...]
    )
    cw12 = (
        jnp.dot(g, wx2e_ref[...], preferred_element_type=jnp.float32)
        + bx2_ref[...]
    )

    w_rel = rel * cw12
    agg12 = jnp.sum(w_rel.reshape(g8, BI, nq, p3), axis=(0, 2))
    agg = jnp.dot(agg12, agfold_ref[...],
                  preferred_element_type=jnp.float32) / (n - 1)

    ms128 = jnp.sum(mf.reshape(g8, BI, nq, pm), axis=(0, 2))
    msum = jnp.dot(ms128, msfold_ref[...],
                   preferred_element_type=jnp.float32)

    vscale = (
        jnp.dot(fi, wv_ref[...], preferred_element_type=jnp.float32)
        + bv_ref[...]
    )
    vnew = vscale * vi + agg
    cnew = ci + vnew

    hpre = _silu(
        jnp.dot(fi, wh1a_ref[...], preferred_element_type=jnp.float32)
        + jnp.dot(msum, wh1b_ref[...], preferred_element_type=jnp.float32)
        + bh1_ref[...]
    )
    hout_ref[...] = (
        fi
        + jnp.dot(hpre, wh2_ref[...], preferred_element_type=jnp.float32)
        + bh2_ref[...]
    )
    cout_ref[...] = _selu(cnew)
    vout_ref[...] = _selu(vnew)


@jax.jit
def kernel(feats, coors, vel, edges, We1, be1, We2, be2, Wx1, bx1, Wx2, bx2,
           Wv, bv, Wh1, bh1, Wh2, bh2):
    b, n, d = feats.shape
    ed = edges.shape[-1]
    h = We1.shape[1]
    m = We2.shape[1]
    np4 = n // P
    g8 = 128 // (P * ed)
    nq = n // (P * g8)

    fn = feats[0]
    cn = coors[0]
    v2 = vel[0]
    fp = (feats[0].reshape(nq, g8, P * d)
          .transpose(1, 0, 2).reshape(np4, P * d))
    cp = (coors[0].reshape(nq, g8, P * 3)
          .transpose(1, 0, 2).reshape(np4, P * 3))
    ep = edges[0].reshape(n, nq, g8 * P * ed)

    eyep = jnp.eye(P, dtype=jnp.float32)
    onesp = jnp.ones((P, 1), dtype=jnp.float32)

    whi = We1[:d]
    whjb = jnp.kron(eyep, We1[d:2 * d])
    wd2 = We1[2 * d:2 * d + 1]
    wd = 0.5 * jnp.kron(eyep, jnp.ones((3, 1), jnp.float32) @ wd2)
    web = jnp.kron(eyep, We1[2 * d + 1:])
    webs = jnp.stack([
        jnp.pad(web, ((g * P * ed, (g8 - 1 - g) * P * ed), (0, 0)))
        for g in range(g8)
    ]) * 0.5
    w2b = 0.5 * jnp.kron(eyep, We2)
    wx1b = 0.5 * jnp.kron(eyep, Wx1)
    wx2e = jnp.kron(eyep, Wx2 @ jnp.ones((1, 3), jnp.float32))
    wh1a = Wh1[:d]
    wh1b = Wh1[d:]
    msfold = jnp.kron(onesp, jnp.eye(m, dtype=jnp.float32))
    agfold = jnp.kron(onesp, jnp.eye(3, dtype=jnp.float32))

    be1r = be1.reshape(1, h)
    be2t = 0.5 * jnp.concatenate([be2.reshape(1, m)] * P, axis=1)
    bx1t = 0.5 * jnp.concatenate([bx1.reshape(1, h)] * P, axis=1)
    bx2r = bx2.reshape(1, 1)
    bvr = bv.reshape(1, 1)
    bh1r = bh1.reshape(1, h)
    bh2r = bh2.reshape(1, d)

    full = lambda shape: pl.BlockSpec(shape, lambda i: tuple(0 for _ in shape))

    grid = (n // BI,)
    out = pl.pallas_call(
        functools.partial(_egnn_step, n=n, d=d, h=h, m=m),
        grid=grid,
        in_specs=[
            full((n, d)),
            full((np4, P * d)),
            full((n, 3)),
            full((np4, P * 3)),
            pl.BlockSpec((BI, 3), lambda i: (i, 0)),
            pl.BlockSpec((BI, nq, g8 * P * ed), lambda i: (i, 0, 0)),
            full((d, h)), full((P * d, P * h)), full((1, h)),
            full((P * 3, P * h)), full((g8, P * ed * g8, P * h)),
            full((P * h, P * m)), full((1, P * m)),
            full((P * m, P * h)), full((1, P * h)),
            full((P * h, P * 3)), full((1, 1)),
            full((d, 1)), full((1, 1)),
            full((d, h)), full((m, h)), full((1, h)), full((h, d)),
            full((1, d)),
            full((P * m, m)), full((P * 3, 3)),
        ],
        out_specs=[
            pl.BlockSpec((BI, d), lambda i: (i, 0)),
            pl.BlockSpec((BI, 3), lambda i: (i, 0)),
            pl.BlockSpec((BI, 3), lambda i: (i, 0)),
        ],
        out_shape=[
            jax.ShapeDtypeStruct((n, d), jnp.float32),
            jax.ShapeDtypeStruct((n, 3), jnp.float32),
            jax.ShapeDtypeStruct((n, 3), jnp.float32),
        ],
        scratch_shapes=[
            pltpu.VMEM((n, h), jnp.float32),
            pltpu.VMEM((np4, P * h), jnp.float32),
        ],
        compiler_params=pltpu.CompilerParams(
            dimension_semantics=("arbitrary",),
        ),
    )(fn, fp, cn, cp, v2, ep,
      whi, whjb, be1r, wd, webs,
      w2b, be2t, wx1b, bx1t, wx2e, bx2r,
      Wv, bvr, wh1a, wh1b, bh1r, Wh2, bh2r,
      msfold, agfold)

    h_out, c_out, v_out = out
    return (h_out[None], c_out[None], v_out[None])
